# baseline (speedup 1.0000x reference)
import functools

import jax
import jax.numpy as jnp
from jax import lax
from jax.experimental import pallas as pl
from jax.experimental.pallas import tpu as pltpu
from jax.experimental.pallas import tpu_sc as plsc

_K = 64


def _tc_body(x_ref, cls_ref, pos_ref, out_ref):
    BB, P, D = x_ref.shape
    for i in range(BB):
        out_ref[i, 0:1, 0:D] = cls_ref[...]
        out_ref[i, 1 : P + 1, 0:D] = x_ref[i]
        out_ref[i, :, D:] = pos_ref[...]


def _tc_kernel(x, cls_embedding, pos):
    B, P, D = x.shape
    E = pos.shape[1]
    bb = 4 if B % 4 == 0 else 1
    return pl.pallas_call(
        _tc_body,
        grid=(B // bb,),
        in_specs=[
            pl.BlockSpec((bb, P, D), lambda b: (b, 0, 0)),
            pl.BlockSpec((1, D), lambda b: (0, 0)),
            pl.BlockSpec((P + 1, E), lambda b: (0, 0)),
        ],
        out_specs=pl.BlockSpec((bb, P + 1, D + E), lambda b: (b, 0, 0)),
        out_shape=jax.ShapeDtypeStruct((B, P + 1, D + E), x.dtype),
    )(x, cls_embedding, pos)


def kernel(x, cls_embedding, pos_embedding_global, pos_embedding_local):
    B, P, D = x.shape
    pos = pos_embedding_global if P == 576 else pos_embedding_local
    E = pos.shape[1]
    R = P + 1
    if (R - 1) % _K != 0 or D % 16 != 0:
        return _tc_kernel(x, cls_embedding, pos)
    x2 = x.reshape(B * P, D)
    info = plsc.get_sparse_core_info()
    nc = info.num_cores
    nw = nc * info.num_subcores
    per = -(-B // nw)
    nch = (R - 1) // _K

    @functools.partial(
        pl.kernel,
        out_type=jax.ShapeDtypeStruct((B, R, D + E), x.dtype),
        mesh=plsc.VectorSubcoreMesh(core_axis_name="c", subcore_axis_name="s"),
        scratch_types=[
            pltpu.VMEM((_K, D), x.dtype),
            pltpu.VMEM((_K, D), x.dtype),
            pltpu.VMEM((1, D), x.dtype),
            pltpu.VMEM((1, D), x.dtype),
            pltpu.VMEM((_K,), jnp.int32),
            pltpu.VMEM((_K,), jnp.int32),
            pltpu.VMEM((16,), jnp.int32),
            pltpu.SemaphoreType.DMA,
            pltpu.SemaphoreType.DMA,
            pltpu.SemaphoreType.DMA,
            pltpu.SemaphoreType.DMA,
            pltpu.SemaphoreType.DMA,
            pltpu.SemaphoreType.DMA,
        ],
    )
    def sc_embed(
        x2_hbm, cls_hbm, pos_hbm, out_hbm,
        buf_a, buf_b, cls_v, last_v, idx_a, idx_b, idx_l,
        sia, sib, soa, sob, sp, sl,
    ):
        wid = lax.axis_index("s") * nc + lax.axis_index("c")
        bufs = (buf_a, buf_b)
        idxs = (idx_a, idx_b)
        sin = (sia, sib)
        sout = (soa, sob)
        ccls = pltpu.make_async_copy(cls_hbm, cls_v, sl)
        ccls.start()
        ccls.wait()
        for j in range(per):
            b = wid * per + j

            @pl.when(b < B)
            def _():
                base = b * P

                def fill_idx(c):
                    idx_ref = idxs[c % 2]
                    for g in range(_K // 16):
                        v = lax.iota(jnp.int32, 16) + (c * _K + g * 16 - 1)
                        idx_ref[pl.ds(g * 16, 16)] = jnp.clip(v, 0, P - 1) + base

                def mk_gather(c):
                    return pltpu.make_async_copy(
                        x2_hbm.at[pl.ds(base + c * _K, _K)], bufs[c % 2], sin[c % 2]
                    )

                def mk_out(c):
                    return pltpu.make_async_copy(
                        bufs[c % 2],
                        out_hbm.at[b, pl.ds(c * _K, _K), pl.ds(0, D)],
                        sout[c % 2],
                    )

                cpos = pltpu.make_async_copy(
                    pos_hbm, out_hbm.at[b, :, pl.ds(D, E)], sp
                )
                cpos.start()
                idx_l[pl.ds(0, 16)] = jnp.full((16,), base + P - 1, jnp.int32)
                glast = pltpu.make_async_copy(
                    x2_hbm.at[idx_l.at[pl.ds(0, 1)]], last_v, sl
                )
                glast.start()
                olast = pltpu.make_async_copy(
                    last_v, out_hbm.at[b, pl.ds(P, 1), pl.ds(0, D)], sl
                )
                gathers = [mk_gather(c) for c in range(nch)]
                outs = [mk_out(c) for c in range(nch)]
                fill_idx(0)
                gathers[0].start()
                glast.wait()
                olast.start()
                for c in range(nch):
                    gathers[c].wait()
                    if c == 0:
                        for g in range(D // 16):
                            bufs[0][0, pl.ds(g * 16, 16)] = cls_v[0, pl.ds(g * 16, 16)]
                    outs[c].start()
                    if c + 1 < nch:
                        if c >= 1:
                            outs[c - 1].wait()
                        fill_idx(c + 1)
                        gathers[c + 1].start()
                if nch >= 2:
                    outs[nch - 2].wait()
                outs[nch - 1].wait()
                olast.wait()
                cpos.wait()

    return sc_embed(x2, cls_embedding, pos)


# speedup vs baseline: 10.6110x; 10.6110x over previous
import jax
import jax.numpy as jnp
from jax.experimental import pallas as pl
from jax.experimental.pallas import tpu as pltpu

_BB = 4


def _body(x_ref, cls_ref, pos_ref, out_ref):
    BB, P, D = x_ref.shape
    for i in range(BB):
        out_ref[i, 0:1, 0:D] = cls_ref[...]
        out_ref[i, 1 : P + 1, 0:D] = x_ref[i]
        out_ref[i, :, D:] = pos_ref[...]


def kernel(x, cls_embedding, pos_embedding_global, pos_embedding_local):
    B, P, D = x.shape
    pos = pos_embedding_global if P == 576 else pos_embedding_local
    E = pos.shape[1]
    bb = _BB if B % _BB == 0 else 1
    out = pl.pallas_call(
        _body,
        grid=(B // bb,),
        in_specs=[
            pl.BlockSpec((bb, P, D), lambda b: (b, 0, 0)),
            pl.BlockSpec((1, D), lambda b: (0, 0)),
            pl.BlockSpec((P + 1, E), lambda b: (0, 0)),
        ],
        out_specs=pl.BlockSpec((bb, P + 1, D + E), lambda b: (b, 0, 0)),
        out_shape=jax.ShapeDtypeStruct((B, P + 1, D + E), x.dtype),
    )(x, cls_embedding, pos)
    return out


# speedup vs baseline: 10.6209x; 1.0009x over previous
import jax
from jax.experimental import pallas as pl

_BB = 4


def _body(x_ref, cls_ref, pos_ref, out_ref):
    BB, P, D = x_ref.shape
    for i in range(BB):
        out_ref[i, 0:1, 0:D] = cls_ref[...]
        out_ref[i, 1 : P + 1, 0:D] = x_ref[i]
        out_ref[i, :, D:] = pos_ref[...]


def kernel(x, cls_embedding, pos_embedding_global, pos_embedding_local):
    B, P, D = x.shape
    pos = pos_embedding_global if P == 576 else pos_embedding_local
    E = pos.shape[1]
    bb = _BB if B % _BB == 0 else 1
    out = pl.pallas_call(
        _body,
        grid=(B // bb,),
        in_specs=[
            pl.BlockSpec((bb, P, D), lambda b: (b, 0, 0)),
            pl.BlockSpec((1, D), lambda b: (0, 0)),
            pl.BlockSpec((P + 1, E), lambda b: (0, 0)),
        ],
        out_specs=pl.BlockSpec((bb, P + 1, D + E), lambda b: (b, 0, 0)),
        out_shape=jax.ShapeDtypeStruct((B, P + 1, D + E), x.dtype),
    )(x, cls_embedding, pos)
    return out
